# Initial kernel scaffold; baseline (speedup 1.0000x reference)
import jax
import jax.numpy as jnp
from jax.experimental import pallas as pl


def kernel(species, edge_src, edge_dst, distances, switch, d_param, p_param, cs_param, alphas_param):
    raise NotImplementedError("write your pallas kernel here")


# speedup vs baseline: 136.2194x; 136.2194x over previous
import functools

import jax
import jax.numpy as jnp
from jax import lax
from jax.experimental import pallas as pl
from jax.experimental.pallas import tpu as pltpu
from jax.experimental.pallas import tpu_sc as plsc

BOHR = 0.52917721092
INV_BOHR = 1.0 / BOHR
N_NODES = 50000
N_EDGES = 1600000

NC = 2
NS = 16
NW = NC * NS
CHUNK_ROWS = 8
ROW = 128
CHUNK = CHUNK_ROWS * ROW
N_CHUNKS = 50
PER_W = CHUNK * N_CHUNKS
E_PAD = PER_W * NW
ACC_PAD = 50176
SLICE = ACC_PAD // NS


def _zbl_sc_kernel(ztab_hbm, zptab_hbm, coef_hbm, zeros_hbm,
                   src_hbm, dst_hbm, dist_hbm, sw_hbm,
                   out_hbm,
                   ztab, zptab, coef,
                   srcb, dstb, distb, swb, vals, sbuf,
                   acc):
    cid = lax.axis_index("c")
    sid = lax.axis_index("s")
    wid = sid * NC + cid

    pltpu.sync_copy(ztab_hbm, ztab)
    pltpu.sync_copy(zptab_hbm, zptab)
    pltpu.sync_copy(coef_hbm, coef)

    pltpu.sync_copy(zeros_hbm, sbuf)
    pltpu.sync_copy(sbuf, acc.at[pl.ds(sid * SLICE, SLICE)])
    plsc.subcore_barrier()

    c0 = coef[0, :]
    c1 = coef[1, :]
    c2 = coef[2, :]
    c3 = coef[3, :]
    na0 = coef[4, :]
    na1 = coef[5, :]
    na2 = coef[6, :]
    na3 = coef[7, :]

    row_base = wid * (N_CHUNKS * CHUNK_ROWS)

    def chunk_body(ch, carry):
        r0 = row_base + ch * CHUNK_ROWS
        pltpu.sync_copy(src_hbm.at[pl.ds(r0, CHUNK_ROWS)], srcb)
        pltpu.sync_copy(dst_hbm.at[pl.ds(r0, CHUNK_ROWS)], dstb)
        pltpu.sync_copy(dist_hbm.at[pl.ds(r0, CHUNK_ROWS)], distb)
        pltpu.sync_copy(sw_hbm.at[pl.ds(r0, CHUNK_ROWS)], swb)
        for j in range(CHUNK_ROWS):
            for k in range(ROW // 16):
                sl = pl.ds(k * 16, 16)
                si = srcb[j, sl]
                di = dstb[j, sl]
                zi = plsc.load_gather(ztab, [si])
                zj = plsc.load_gather(ztab, [di])
                zpi = plsc.load_gather(zptab, [si])
                zpj = plsc.load_gather(zptab, [di])
                r = distb[j, sl] * INV_BOHR
                x = r * (zpi + zpj)
                phi = (c0 * jnp.exp(na0 * x) + c1 * jnp.exp(na1 * x)
                       + c2 * jnp.exp(na2 * x) + c3 * jnp.exp(na3 * x))
                vals[j, sl] = zi * zj * phi * swb[j, sl] / r
            pltpu.sync_copy(vals.at[j], acc.at[srcb.at[j]], add=True)
        return carry

    lax.fori_loop(0, N_CHUNKS, chunk_body, None)

    plsc.subcore_barrier()
    pltpu.sync_copy(acc.at[pl.ds(sid * SLICE, SLICE)], sbuf)
    pltpu.sync_copy(sbuf,
                    out_hbm.at[pl.ds(cid * ACC_PAD + sid * SLICE, SLICE)])


_zbl_sc = functools.partial(
    pl.kernel,
    out_type=jax.ShapeDtypeStruct((NC * ACC_PAD,), jnp.float32),
    mesh=plsc.VectorSubcoreMesh(core_axis_name="c", subcore_axis_name="s"),
    compiler_params=pltpu.CompilerParams(needs_layout_passes=False),
    scratch_types=[
        pltpu.VMEM((N_NODES,), jnp.float32),
        pltpu.VMEM((N_NODES,), jnp.float32),
        pltpu.VMEM((8, 16), jnp.float32),
        pltpu.VMEM((CHUNK_ROWS, ROW), jnp.int32),
        pltpu.VMEM((CHUNK_ROWS, ROW), jnp.int32),
        pltpu.VMEM((CHUNK_ROWS, ROW), jnp.float32),
        pltpu.VMEM((CHUNK_ROWS, ROW), jnp.float32),
        pltpu.VMEM((CHUNK_ROWS, ROW), jnp.float32),
        pltpu.VMEM((SLICE,), jnp.float32),
        pltpu.VMEM_SHARED((ACC_PAD,), jnp.float32),
    ],
)(_zbl_sc_kernel)


def _combine_body(p_ref, o_ref):
    o_ref[...] = p_ref[0] + p_ref[1]


def _combine(partials):
    return pl.pallas_call(
        _combine_body,
        out_shape=jax.ShapeDtypeStruct((ACC_PAD // 128, 128), jnp.float32),
    )(partials.reshape(NC, ACC_PAD // 128, 128))


def kernel(species, edge_src, edge_dst, distances, switch,
           d_param, p_param, cs_param, alphas_param):
    f32 = jnp.float32
    d = jnp.abs(d_param).astype(f32)
    p = jnp.abs(p_param).astype(f32)
    cs = 0.5 * jax.nn.softmax(cs_param.astype(f32))
    alphas = jnp.abs(alphas_param).astype(f32)

    Z = jnp.where(species > 0, species.astype(f32), 0.0)
    Zp = Z ** p / d

    coef = jnp.broadcast_to(
        jnp.concatenate([cs, -alphas])[:, None], (8, 16)).astype(f32)
    zeros = jnp.zeros((SLICE,), f32)

    pad = E_PAD - N_EDGES
    src = jnp.concatenate(
        [edge_src.astype(jnp.int32), jnp.zeros((pad,), jnp.int32)]
    ).reshape(-1, ROW)
    dst = jnp.concatenate(
        [edge_dst.astype(jnp.int32), jnp.zeros((pad,), jnp.int32)]
    ).reshape(-1, ROW)
    dist = jnp.concatenate(
        [distances.astype(f32), jnp.ones((pad,), f32)]
    ).reshape(-1, ROW)
    sw = jnp.concatenate(
        [switch.astype(f32), jnp.zeros((pad,), f32)]
    ).reshape(-1, ROW)

    partials = _zbl_sc(Z, Zp, coef, zeros, src, dst, dist, sw)
    summed = _combine(partials)
    return summed.reshape(-1)[:N_NODES]


# speedup vs baseline: 203.3552x; 1.4928x over previous
import functools

import jax
import jax.numpy as jnp
from jax import lax
from jax.experimental import pallas as pl
from jax.experimental.pallas import tpu as pltpu
from jax.experimental.pallas import tpu_sc as plsc

BOHR = 0.52917721092
INV_BOHR = 1.0 / BOHR
N_NODES = 50000
N_EDGES = 1600000

NC = 2
NS = 16
NW = NC * NS
CHUNK_ROWS = 8
ROW = 128
CHUNK = CHUNK_ROWS * ROW
N_CHUNKS = 50
PER_W = CHUNK * N_CHUNKS
E_PAD = PER_W * NW
ROWS_TOTAL = E_PAD // ROW
ROWS_ALLOC = ROWS_TOTAL + CHUNK_ROWS
ACC_PAD = 50176
SLICE = ACC_PAD // NS


def _zbl_sc_kernel(ztab_hbm, zptab_hbm, coef_hbm, zeros_hbm,
                   src_hbm, dst_hbm, dist_hbm, sw_hbm,
                   out_hbm,
                   ztab, zptab, coef,
                   srcb, dstb, distb, swb, vals, sbuf,
                   acc,
                   sem_in, sem_sc):
    cid = lax.axis_index("c")
    sid = lax.axis_index("s")
    wid = sid * NC + cid

    pltpu.sync_copy(ztab_hbm, ztab)
    pltpu.sync_copy(zptab_hbm, zptab)
    pltpu.sync_copy(coef_hbm, coef)

    pltpu.sync_copy(zeros_hbm, sbuf)
    pltpu.sync_copy(sbuf, acc.at[pl.ds(sid * SLICE, SLICE)])
    plsc.subcore_barrier()

    c0 = coef[0, :]
    c1 = coef[1, :]
    c2 = coef[2, :]
    c3 = coef[3, :]
    na0 = coef[4, :]
    na1 = coef[5, :]
    na2 = coef[6, :]
    na3 = coef[7, :]

    row_base = wid * (N_CHUNKS * CHUNK_ROWS)

    def fire_inputs(ch, slot):
        r0 = row_base + ch * CHUNK_ROWS
        rs = pl.ds(r0, CHUNK_ROWS)
        pltpu.async_copy(src_hbm.at[rs], srcb.at[slot], sem_in.at[slot])
        pltpu.async_copy(dst_hbm.at[rs], dstb.at[slot], sem_in.at[slot])
        pltpu.async_copy(dist_hbm.at[rs], distb.at[slot], sem_in.at[slot])
        pltpu.async_copy(sw_hbm.at[rs], swb.at[slot], sem_in.at[slot])

    def wait_inputs(ch, slot):
        r0 = row_base + ch * CHUNK_ROWS
        rs = pl.ds(r0, CHUNK_ROWS)
        pltpu.make_async_copy(src_hbm.at[rs], srcb.at[slot], sem_in.at[slot]).wait()
        pltpu.make_async_copy(dst_hbm.at[rs], dstb.at[slot], sem_in.at[slot]).wait()
        pltpu.make_async_copy(dist_hbm.at[rs], distb.at[slot], sem_in.at[slot]).wait()
        pltpu.make_async_copy(sw_hbm.at[rs], swb.at[slot], sem_in.at[slot]).wait()

    def compute_chunk(slot):
        prev = None
        for j in range(CHUNK_ROWS):
            for k in range(ROW // 16):
                sl = pl.ds(k * 16, 16)
                si = srcb[slot, j, sl]
                di = dstb[slot, j, sl]
                zi = plsc.load_gather(ztab, [si])
                zj = plsc.load_gather(ztab, [di])
                zpi = plsc.load_gather(zptab, [si])
                zpj = plsc.load_gather(zptab, [di])
                r = distb[slot, j, sl] * INV_BOHR
                x = r * (zpi + zpj)
                phi = (c0 * jnp.exp(na0 * x) + c1 * jnp.exp(na1 * x)
                       + c2 * jnp.exp(na2 * x) + c3 * jnp.exp(na3 * x))
                vals[slot, j, sl] = zi * zj * phi * swb[slot, j, sl] / r
            if prev is not None:
                prev.wait()
            prev = pltpu.async_copy(vals.at[slot, j], acc.at[srcb.at[slot, j]],
                                    sem_sc.at[slot], add=True)
        prev.wait()

    fire_inputs(0, 0)

    def pair_body(i, carry):
        ch0 = i * 2
        fire_inputs(ch0 + 1, 1)
        wait_inputs(ch0, 0)
        compute_chunk(0)
        fire_inputs(ch0 + 2, 0)
        wait_inputs(ch0 + 1, 1)
        compute_chunk(1)
        return carry

    lax.fori_loop(0, N_CHUNKS // 2, pair_body, None)
    wait_inputs(N_CHUNKS, 0)

    plsc.subcore_barrier()
    pltpu.sync_copy(acc.at[pl.ds(sid * SLICE, SLICE)], sbuf)
    pltpu.sync_copy(sbuf,
                    out_hbm.at[pl.ds(cid * ACC_PAD + sid * SLICE, SLICE)])


_zbl_sc = functools.partial(
    pl.kernel,
    out_type=jax.ShapeDtypeStruct((NC * ACC_PAD,), jnp.float32),
    mesh=plsc.VectorSubcoreMesh(core_axis_name="c", subcore_axis_name="s"),
    compiler_params=pltpu.CompilerParams(needs_layout_passes=False),
    scratch_types=[
        pltpu.VMEM((N_NODES,), jnp.float32),
        pltpu.VMEM((N_NODES,), jnp.float32),
        pltpu.VMEM((8, 16), jnp.float32),
        pltpu.VMEM((2, CHUNK_ROWS, ROW), jnp.int32),
        pltpu.VMEM((2, CHUNK_ROWS, ROW), jnp.int32),
        pltpu.VMEM((2, CHUNK_ROWS, ROW), jnp.float32),
        pltpu.VMEM((2, CHUNK_ROWS, ROW), jnp.float32),
        pltpu.VMEM((2, CHUNK_ROWS, ROW), jnp.float32),
        pltpu.VMEM((SLICE,), jnp.float32),
        pltpu.VMEM_SHARED((ACC_PAD,), jnp.float32),
        pltpu.SemaphoreType.DMA((2,)),
        pltpu.SemaphoreType.DMA((2,)),
    ],
)(_zbl_sc_kernel)


def _combine_body(p_ref, o_ref):
    o_ref[...] = p_ref[0] + p_ref[1]


def _combine(partials):
    return pl.pallas_call(
        _combine_body,
        out_shape=jax.ShapeDtypeStruct((ACC_PAD // 128, 128), jnp.float32),
    )(partials.reshape(NC, ACC_PAD // 128, 128))


def kernel(species, edge_src, edge_dst, distances, switch,
           d_param, p_param, cs_param, alphas_param):
    f32 = jnp.float32
    d = jnp.abs(d_param).astype(f32)
    p = jnp.abs(p_param).astype(f32)
    cs = 0.5 * jax.nn.softmax(cs_param.astype(f32))
    alphas = jnp.abs(alphas_param).astype(f32)

    Z = jnp.where(species > 0, species.astype(f32), 0.0)
    Zp = Z ** p / d

    coef = jnp.broadcast_to(
        jnp.concatenate([cs, -alphas])[:, None], (8, 16)).astype(f32)
    zeros = jnp.zeros((SLICE,), f32)

    pad = ROWS_ALLOC * ROW - N_EDGES
    src = jnp.concatenate(
        [edge_src.astype(jnp.int32), jnp.zeros((pad,), jnp.int32)]
    ).reshape(-1, ROW)
    dst = jnp.concatenate(
        [edge_dst.astype(jnp.int32), jnp.zeros((pad,), jnp.int32)]
    ).reshape(-1, ROW)
    dist = jnp.concatenate(
        [distances.astype(f32), jnp.ones((pad,), f32)]
    ).reshape(-1, ROW)
    sw = jnp.concatenate(
        [switch.astype(f32), jnp.zeros((pad,), f32)]
    ).reshape(-1, ROW)

    partials = _zbl_sc(Z, Zp, coef, zeros, src, dst, dist, sw)
    summed = _combine(partials)
    return summed.reshape(-1)[:N_NODES]
